# baseline (speedup 1.0000x reference)
import functools

import jax
import jax.numpy as jnp
from jax import lax
from jax.experimental import pallas as pl
from jax.experimental.pallas import tpu as pltpu
from jax.experimental.pallas import tpu_sc as plsc

NUM_ENTITIES = 1000000
NUM_RELATIONS = 1000
D = 32
B = 16384
NC = 2
NS = 16
L = 16
NW = NC * NS
BPW = B // NW
CH = 128
NCH = BPW // CH

_mesh = plsc.VectorSubcoreMesh(core_axis_name="c", subcore_axis_name="s")


@functools.partial(
    pl.kernel,
    out_type=jax.ShapeDtypeStruct((B,), jnp.float32),
    mesh=_mesh,
    compiler_params=pltpu.CompilerParams(
        use_tc_tiling_on_sc=False,
        needs_layout_passes=False,
    ),
    scratch_types=[
        pltpu.VMEM((NCH, CH), jnp.int32),
        pltpu.VMEM((NCH, CH), jnp.int32),
        pltpu.VMEM((NCH, CH), jnp.int32),
        pltpu.VMEM((BPW, D), jnp.float32),
        pltpu.VMEM((BPW, D), jnp.float32),
        pltpu.VMEM((BPW, D), jnp.float32),
        pltpu.VMEM((BPW,), jnp.float32),
        pltpu.SemaphoreType.DMA,
    ],
)
def _transe_scores(ent_h, rel_h, hi_h, ri_h, ti_h, out_h,
                   hidx, ridx, tidx, hrows, rrows, trows, outv, sem):
    wid = lax.axis_index("s") * NC + lax.axis_index("c")
    base = wid * BPW

    for q in range(NCH):
        off = base + q * CH
        pltpu.sync_copy(hi_h.at[pl.ds(off, CH)], hidx.at[q])
        pltpu.sync_copy(ri_h.at[pl.ds(off, CH)], ridx.at[q])
        pltpu.sync_copy(ti_h.at[pl.ds(off, CH)], tidx.at[q])

    copies = []
    for q in range(NCH):
        dst = pl.ds(q * CH, CH)
        copies.append(pltpu.async_copy(ent_h.at[hidx.at[q]], hrows.at[dst], sem))
        copies.append(pltpu.async_copy(ent_h.at[tidx.at[q]], trows.at[dst], sem))
        copies.append(pltpu.async_copy(rel_h.at[ridx.at[q]], rrows.at[dst], sem))
    for c in copies:
        c.wait()

    lanes = lax.iota(jnp.int32, L)

    def group_body(g, carry):
        rbase = g * L
        acc = jnp.zeros((L,), jnp.float32)
        for k in range(L):
            row = rbase + k
            h0 = hrows[row, pl.ds(0, L)]
            h1 = hrows[row, pl.ds(L, L)]
            r0 = rrows[row, pl.ds(0, L)]
            r1 = rrows[row, pl.ds(L, L)]
            t0 = trows[row, pl.ds(0, L)]
            t1 = trows[row, pl.ds(L, L)]
            d0 = (h0 + r0) - t0
            d1 = (h1 + r1) - t1
            s = d0 * d0 + d1 * d1
            acc = jnp.where(lanes == k, jnp.sum(s), acc)
        i32 = plsc.bitcast(acc, jnp.int32)
        y = plsc.bitcast(jnp.full((L,), 0x5F3759DF, jnp.int32) - (i32 >> 1),
                         jnp.float32)
        for _ in range(3):
            y = y * (1.5 - 0.5 * ((acc * y) * y))
        outv[pl.ds(rbase, L)] = -(acc * y)
        return carry

    lax.fori_loop(0, BPW // L, group_body, 0)

    pltpu.sync_copy(outv, out_h.at[pl.ds(base, BPW)])


def kernel(entity_embeddings, relation_embeddings,
           head_indices, relation_indices, tail_indices):
    return _transe_scores(entity_embeddings, relation_embeddings,
                          head_indices, relation_indices, tail_indices)
